# baseline (speedup 1.0000x reference)
import functools

import jax
import jax.numpy as jnp
import numpy as np
from jax import lax
from jax.experimental import pallas as pl
from jax.experimental.pallas import tpu as pltpu
from jax.experimental.pallas import tpu_sc as plsc

PITCH_VOCAB = 100000
DUR_VOCAB = 1000
D = 128
B, L = 4096, 200
N = B * L

NC, NS, LANES = 2, 16, 16
NW = NC * NS
TOK_PER_W = N // NW
C = 128
NCHUNK = TOK_PER_W // C
SB = 40
NSUPER = NCHUNK // SB
SCALE = float(np.sqrt(np.float32(D)))
MASK_HI = -65536

_mesh = plsc.VectorSubcoreMesh(core_axis_name="c", subcore_axis_name="s")


@functools.partial(
    pl.kernel,
    out_type=jax.ShapeDtypeStruct((N, D), jnp.float32),
    mesh=_mesh,
    compiler_params=pltpu.CompilerParams(needs_layout_passes=False),
    scratch_types=[
        pltpu.VMEM((DUR_VOCAB * D // 2,), jnp.int32),
        pltpu.VMEM((2, SB, C), jnp.int32),
        pltpu.VMEM((2, SB, C), jnp.int32),
        pltpu.VMEM((2, C, D), jnp.float32),
        pltpu.SemaphoreType.DMA,
        pltpu.SemaphoreType.DMA,
        pltpu.SemaphoreType.DMA,
        pltpu.SemaphoreType.DMA,
        pltpu.SemaphoreType.DMA,
        pltpu.SemaphoreType.DMA,
    ],
)
def _dual_embed(ptok, dtok, dtabw, ptab, out, dtab_v, idx_p, idx_d, rows,
                sem_t, sem_i, sg0, sg1, so0, so1):
    wid = lax.axis_index("s") * NC + lax.axis_index("c")
    base0 = wid * TOK_PER_W
    sgs = (sg0, sg1)
    sos = (so0, so1)

    pltpu.async_copy(dtabw, dtab_v, sem_t)

    def issue_idx(s, sb):
        pltpu.async_copy(ptok.at[wid, pl.ds(s * SB, SB)], idx_p.at[sb], sem_i)
        pltpu.async_copy(dtok.at[wid, pl.ds(s * SB, SB)], idx_d.at[sb], sem_i)

    def wait_idx(s, sb):
        pltpu.make_async_copy(
            ptok.at[wid, pl.ds(s * SB, SB)], idx_p.at[sb], sem_i
        ).wait()
        pltpu.make_async_copy(
            dtok.at[wid, pl.ds(s * SB, SB)], idx_d.at[sb], sem_i
        ).wait()

    def issue_gather(sb, j, b):
        pltpu.async_copy(ptab.at[idx_p.at[sb, j]], rows.at[b], sgs[b])

    def wait_gather(sb, j, b):
        pltpu.make_async_copy(ptab.at[idx_p.at[sb, j]], rows.at[b], sgs[b]).wait()

    issue_idx(0, 0)
    wait_idx(0, 0)
    pltpu.make_async_copy(dtabw, dtab_v, sem_t).wait()
    issue_gather(0, 0, 0)

    def chunk_step(g, b):
        s = g // SB
        j = lax.rem(g, SB)
        sb = lax.rem(s, 2)

        @pl.when((j == 0) & (s + 1 < NSUPER))
        def _():
            issue_idx(s + 1, 1 - sb)

        @pl.when(j + 1 < SB)
        def _():
            issue_gather(sb, j + 1, 1 - b)

        @pl.when((j + 1 == SB) & (s + 1 < NSUPER))
        def _():
            wait_idx(s + 1, 1 - sb)
            issue_gather(1 - sb, 0, 1 - b)

        @pl.when(g >= 2)
        def _():
            pltpu.make_async_copy(
                rows.at[b], out.at[pl.ds(base0 + (g - 2) * C, C)], sos[b]
            ).wait()

        wait_gather(sb, j, b)
        rp = rows.at[b]

        def row_body(rr, c2):
            dvec = idx_d[sb, j, pl.ds(LANES * rr, LANES)]
            for i in range(LANES):
                did = dvec[i]
                r = LANES * rr + i
                dbase = pl.multiple_of(did * (D // 2), 16)
                for k in range(D // 32):
                    w = dtab_v[pl.ds(dbase + 16 * k, 16)]
                    fa = plsc.bitcast(w << 16, jnp.float32)
                    fb = plsc.bitcast(w & MASK_HI, jnp.float32)
                    sl0 = pl.ds(32 * k, LANES)
                    sl1 = pl.ds(32 * k + LANES, LANES)
                    rp[r, sl0] = SCALE * (rp[r, sl0] + fa)
                    rp[r, sl1] = SCALE * (rp[r, sl1] + fb)
            return c2

        lax.fori_loop(0, C // LANES, row_body, 0)
        pltpu.async_copy(rp, out.at[pl.ds(base0 + g * C, C)], sos[b])

    def pair_body(g2, carry):
        for b in range(2):
            chunk_step(2 * g2 + b, b)
        return carry

    lax.fori_loop(0, NCHUNK // 2, pair_body, 0)

    pltpu.make_async_copy(
        rows.at[0], out.at[pl.ds(base0 + (NCHUNK - 2) * C, C)], so0
    ).wait()
    pltpu.make_async_copy(
        rows.at[1], out.at[pl.ds(base0 + (NCHUNK - 1) * C, C)], so1
    ).wait()


def _pack_duration(duration_table):
    dt = duration_table.astype(jnp.bfloat16)
    du = jax.lax.bitcast_convert_type(dt, jnp.uint16).astype(jnp.uint32)
    dr = du.reshape(DUR_VOCAB, D // 32, 2, LANES)
    words = dr[:, :, 0, :] | (dr[:, :, 1, :] << 16)
    return jax.lax.bitcast_convert_type(words, jnp.int32).reshape(DUR_VOCAB * D // 2)


def kernel(pitch_tokens, duration_tokens, pitch_table, duration_table):
    out = _dual_embed(
        pitch_tokens.reshape(NW, NCHUNK, C).astype(jnp.int32),
        duration_tokens.reshape(NW, NCHUNK, C).astype(jnp.int32),
        _pack_duration(duration_table),
        pitch_table,
    )
    return out.reshape(B, L, D)


# speedup vs baseline: 1.3649x; 1.3649x over previous
import functools

import jax
import jax.numpy as jnp
import numpy as np
from jax import lax
from jax.experimental import pallas as pl
from jax.experimental.pallas import tpu as pltpu
from jax.experimental.pallas import tpu_sc as plsc

PITCH_VOCAB = 100000
DUR_VOCAB = 1000
D = 128
B, L = 4096, 200
N = B * L

NC, NS, LANES = 2, 16, 16
NW = NC * NS
TOK_PER_W = N // NW
C = 128
NCHUNK = TOK_PER_W // C
SCALE = float(np.sqrt(np.float32(D)))

_mesh = plsc.VectorSubcoreMesh(core_axis_name="c", subcore_axis_name="s")


@functools.partial(
    pl.kernel,
    out_type=jax.ShapeDtypeStruct((N, D), jnp.float32),
    mesh=_mesh,
    scratch_types=[
        pltpu.VMEM((NCHUNK, C), jnp.int32),
        pltpu.VMEM((NCHUNK, C), jnp.int32),
        pltpu.VMEM((2, C, D), jnp.float32),
        pltpu.VMEM((2, C, D), jnp.float32),
        pltpu.SemaphoreType.DMA,
        pltpu.SemaphoreType.DMA,
        pltpu.SemaphoreType.DMA,
        pltpu.SemaphoreType.DMA,
        pltpu.SemaphoreType.DMA,
    ],
)
def _dual_embed(ptok, dtok, ptab, dtab, out, idx_p, idx_d, rows_p, rows_d,
                sem_i, sg0, sg1, so0, so1):
    wid = lax.axis_index("s") * NC + lax.axis_index("c")
    base0 = wid * TOK_PER_W
    sgs = (sg0, sg1)
    sos = (so0, so1)

    cp1 = pltpu.async_copy(ptok.at[wid], idx_p, sem_i)
    cp2 = pltpu.async_copy(dtok.at[wid], idx_d, sem_i)
    cp1.wait()
    cp2.wait()

    def issue_gather(g, b):
        pltpu.async_copy(ptab.at[idx_p.at[g]], rows_p.at[b], sgs[b])
        pltpu.async_copy(dtab.at[idx_d.at[g]], rows_d.at[b], sgs[b])

    def wait_gather(g, b):
        pltpu.make_async_copy(ptab.at[idx_p.at[g]], rows_p.at[b], sgs[b]).wait()
        pltpu.make_async_copy(dtab.at[idx_d.at[g]], rows_d.at[b], sgs[b]).wait()

    def compute_and_flush(g, b):
        @pl.when(g >= 2)
        def _():
            pltpu.make_async_copy(
                rows_p.at[b], out.at[pl.ds(base0 + (g - 2) * C, C)], sos[b]
            ).wait()

        wait_gather(g, b)
        rp = rows_p.at[b]
        rd = rows_d.at[b]

        def row_body(r, c2):
            for k in range(D // LANES):
                sl = pl.ds(k * LANES, LANES)
                rp[r, sl] = SCALE * (rp[r, sl] + rd[r, sl])
            return c2

        lax.fori_loop(0, C, row_body, 0)
        pltpu.async_copy(rp, out.at[pl.ds(base0 + g * C, C)], sos[b])

    issue_gather(0, 0)

    def outer(g2, carry):
        for b in range(2):
            g = 2 * g2 + b

            @pl.when(g + 1 < NCHUNK)
            def _():
                issue_gather(g + 1, 1 - b)

            compute_and_flush(g, b)
        return carry

    lax.fori_loop(0, NCHUNK // 2, outer, 0)

    pltpu.make_async_copy(
        rows_p.at[0], out.at[pl.ds(base0 + (NCHUNK - 2) * C, C)], so0
    ).wait()
    pltpu.make_async_copy(
        rows_p.at[1], out.at[pl.ds(base0 + (NCHUNK - 1) * C, C)], so1
    ).wait()


def kernel(pitch_tokens, duration_tokens, pitch_table, duration_table):
    out = _dual_embed(
        pitch_tokens.reshape(NW, NCHUNK, C).astype(jnp.int32),
        duration_tokens.reshape(NW, NCHUNK, C).astype(jnp.int32),
        pitch_table,
        duration_table,
    )
    return out.reshape(B, L, D)
